# baseline (speedup 1.0000x reference)
import functools

import jax
import jax.numpy as jnp
from jax import lax
from jax.experimental import pallas as pl
from jax.experimental.pallas import tpu as pltpu
from jax.experimental.pallas import tpu_sc as plsc

_NUM_CORES = 2
_NUM_SUBCORES = 16
_NW = _NUM_CORES * _NUM_SUBCORES
_CH = 128


_SG = 8


@functools.lru_cache(maxsize=None)
def _make_gather(n_rows: int, d: int):
  assert n_rows % (_NW * _CH) == 0
  k = n_rows // (_NW * _CH)
  assert k % _SG == 0
  n_steps = k // _SG
  mesh = plsc.VectorSubcoreMesh(
      core_axis_name="c", subcore_axis_name="s",
      num_cores=_NUM_CORES, num_subcores=_NUM_SUBCORES)

  @functools.partial(
      pl.kernel,
      mesh=mesh,
      out_type=jax.ShapeDtypeStruct((n_rows, d), jnp.float32),
      compiler_params=pltpu.CompilerParams(use_tc_tiling_on_sc=False),
      scratch_types=[
          pltpu.VMEM((k * _CH,), jnp.int32),
          pltpu.VMEM((_SG * _CH, d), jnp.float32),
          pltpu.VMEM((_SG * _CH, d), jnp.float32),
          pltpu.SemaphoreType.DMA,
          pltpu.SemaphoreType.DMA,
          pltpu.SemaphoreType.DMA,
          pltpu.SemaphoreType.DMA,
      ],
  )
  def gather_kernel(idx_hbm, table_hbm, out_hbm, idx_v, buf0, buf1, gsem0,
                    gsem1, wsem0, wsem1):
    wid = lax.axis_index("s") * _NUM_CORES + lax.axis_index("c")
    pltpu.sync_copy(idx_hbm.at[pl.ds(wid * (k * _CH), k * _CH)], idx_v)
    bufs = (buf0, buf1)
    gsems = (gsem0, gsem1)
    wsems = (wsem0, wsem1)
    rows_per_stream = _SG * _CH
    base = wid * k * _CH

    g = [None] * n_steps
    w = [None] * n_steps
    for s in range(n_steps):
      if s >= 2:
        w[s - 2].wait()
      g[s] = pltpu.async_copy(
          table_hbm.at[idx_v.at[pl.ds(s * rows_per_stream, rows_per_stream)]],
          bufs[s % 2], gsems[s % 2])
      if s >= 1:
        g[s - 1].wait()
        w[s - 1] = pltpu.async_copy(
            bufs[(s - 1) % 2],
            out_hbm.at[pl.ds(base + (s - 1) * rows_per_stream,
                             rows_per_stream)],
            wsems[(s - 1) % 2])
    last = n_steps - 1
    g[last].wait()
    w[last] = pltpu.async_copy(
        bufs[last % 2],
        out_hbm.at[pl.ds(base + last * rows_per_stream, rows_per_stream)],
        wsems[last % 2])
    w[last - 1].wait()
    w[last].wait()

  return gather_kernel


def kernel(x, table):
  b, f = x.shape
  v, d = table.shape
  n = b * f
  idx = x.reshape(n).astype(jnp.int32)
  out = _make_gather(n, d)(idx, table)
  return out.reshape(b, f, d)


# speedup vs baseline: 1.0581x; 1.0581x over previous
import functools

import jax
import jax.numpy as jnp
from jax import lax
from jax.experimental import pallas as pl
from jax.experimental.pallas import tpu as pltpu
from jax.experimental.pallas import tpu_sc as plsc

_NUM_CORES = 2
_NUM_SUBCORES = 16
_NW = _NUM_CORES * _NUM_SUBCORES


@functools.lru_cache(maxsize=None)
def _make_gather(f: int, b: int, d: int):
  bw = b // _NW
  assert b % _NW == 0
  mesh = plsc.VectorSubcoreMesh(
      core_axis_name="c", subcore_axis_name="s",
      num_cores=_NUM_CORES, num_subcores=_NUM_SUBCORES)

  @functools.partial(
      pl.kernel,
      mesh=mesh,
      out_type=jax.ShapeDtypeStruct((f, b, d), jnp.float32),
      compiler_params=pltpu.CompilerParams(use_tc_tiling_on_sc=False),
      scratch_types=[
          pltpu.VMEM((f, bw), jnp.int32),
          pltpu.VMEM((bw, d), jnp.float32),
          pltpu.VMEM((bw, d), jnp.float32),
          pltpu.SemaphoreType.DMA,
          pltpu.SemaphoreType.DMA,
          pltpu.SemaphoreType.DMA,
          pltpu.SemaphoreType.DMA,
      ],
  )
  def gather_kernel(xt_hbm, table_hbm, out_hbm, idx_v, buf0, buf1, gsem0,
                    gsem1, wsem0, wsem1):
    wid = lax.axis_index("s") * _NUM_CORES + lax.axis_index("c")
    base = wid * bw
    pltpu.sync_copy(xt_hbm.at[:, pl.ds(base, bw)], idx_v)
    bufs = (buf0, buf1)
    gsems = (gsem0, gsem1)
    wsems = (wsem0, wsem1)

    g = [None] * f
    w = [None] * f
    for s in range(f):
      if s >= 2:
        w[s - 2].wait()
      g[s] = pltpu.async_copy(
          table_hbm.at[idx_v.at[s]], bufs[s % 2], gsems[s % 2])
      if s >= 1:
        g[s - 1].wait()
        w[s - 1] = pltpu.async_copy(
            bufs[(s - 1) % 2], out_hbm.at[s - 1, pl.ds(base, bw)],
            wsems[(s - 1) % 2])
    g[f - 1].wait()
    w[f - 1] = pltpu.async_copy(
        bufs[(f - 1) % 2], out_hbm.at[f - 1, pl.ds(base, bw)],
        wsems[(f - 1) % 2])
    w[f - 2].wait()
    w[f - 1].wait()

  return gather_kernel


def kernel(x, table):
  b, f = x.shape
  v, d = table.shape
  xt = x.T.astype(jnp.int32)
  out = _make_gather(f, b, d)(xt, table)
  return jnp.transpose(out, (1, 0, 2))
